# baseline (speedup 1.0000x reference)
import functools

import jax
import jax.numpy as jnp
from jax import lax
from jax.experimental import pallas as pl
from jax.experimental.pallas import tpu as pltpu
from jax.experimental.pallas import tpu_sc as plsc

_LOG2 = 0.6931471805599453

_NC = 2
_NS = 16
_NW = _NC * _NS


def _ssp(x):
    return jax.nn.softplus(x) - _LOG2


def _h_body(x_ref, w_ref, o_ref):
    o_ref[...] = jnp.dot(x_ref[...], w_ref[...],
                         preferred_element_type=jnp.float32)


def _compute_h(x, W_in2f, block=1000):
    n, d = x.shape
    f = W_in2f.shape[1]
    grid = n // block
    return pl.pallas_call(
        _h_body,
        grid=(grid,),
        in_specs=[
            pl.BlockSpec((block, d), lambda i: (i, 0)),
            pl.BlockSpec((d, f), lambda i: (0, 0)),
        ],
        out_specs=pl.BlockSpec((block, f), lambda i: (i, 0)),
        out_shape=jax.ShapeDtypeStruct((n, f), jnp.float32),
    )(x, W_in2f)


def _filter_body(f_ref, rc_ref, w1_ref, b1_ref, w2_ref, b2_ref, o_ref):
    t = jnp.dot(f_ref[...], w1_ref[...], preferred_element_type=jnp.float32)
    t = _ssp(t + b1_ref[...])
    t = jnp.dot(t.astype(jnp.bfloat16), w2_ref[...].astype(jnp.bfloat16),
                preferred_element_type=jnp.float32)
    o_ref[...] = (t + b2_ref[...]) * rc_ref[...]


def _compute_wij(f_ij, rcut, W_f1, b_f1, W_f2, b_f2, e_slab, row_off,
                 block=4000):
    e, r = f_ij.shape
    nf = W_f1.shape[1]
    grid = e_slab // block
    off = row_off // block
    rcut2 = rcut.reshape(e, 1)
    b1 = b_f1.reshape(1, nf)
    b2 = b_f2.reshape(1, nf)
    return pl.pallas_call(
        _filter_body,
        grid=(grid,),
        in_specs=[
            pl.BlockSpec((block, r), lambda i: (i + off, 0)),
            pl.BlockSpec((block, 1), lambda i: (i + off, 0)),
            pl.BlockSpec((r, nf), lambda i: (0, 0)),
            pl.BlockSpec((1, nf), lambda i: (0, 0)),
            pl.BlockSpec((nf, nf), lambda i: (0, 0)),
            pl.BlockSpec((1, nf), lambda i: (0, 0)),
        ],
        out_specs=pl.BlockSpec((block, nf), lambda i: (i, 0)),
        out_shape=jax.ShapeDtypeStruct((e_slab, nf), jnp.float32),
    )(f_ij, rcut2, W_f1, b1, W_f2, b2)


def _make_sc_gather_scatter(n_atoms, n_slab, edge_off, d, chunk=40,
                            nhalf=1):
    epw = n_slab // _NW
    nchunk = epw // chunk
    nch = nchunk // nhalf
    eph = nch * chunk
    assert nchunk == nch * nhalf and nch % 2 == 1 and nch >= 3
    n_pad = ((n_atoms + 8 * _NS - 1) // (8 * _NS)) * (8 * _NS)
    rows_per_tile = n_pad // _NS
    vecs_per_row = d // 16
    mesh = plsc.VectorSubcoreMesh(core_axis_name="c", subcore_axis_name="s")

    @functools.partial(
        pl.kernel,
        out_type=jax.ShapeDtypeStruct((_NC, n_pad, d), jnp.float32),
        mesh=mesh,
        scratch_types=[
            pltpu.VMEM((eph,), jnp.int32),
            pltpu.VMEM((chunk,), jnp.int32),
            pltpu.VMEM((chunk,), jnp.int32),
            pltpu.VMEM((chunk, d), jnp.float32),
            pltpu.VMEM((chunk, d), jnp.float32),
            pltpu.VMEM((chunk, d), jnp.float32),
            pltpu.VMEM((chunk, d), jnp.float32),
            pltpu.VMEM_SHARED((n_pad, d), jnp.float32),
            pltpu.SemaphoreType.DMA,
            pltpu.SemaphoreType.DMA,
            pltpu.SemaphoreType.DMA,
            pltpu.SemaphoreType.DMA,
            pltpu.SemaphoreType.DMA,
            pltpu.SemaphoreType.DMA,
        ],
    )
    def sc_kern(h_hbm, wij_hbm, idxi_hbm, idxj_hbm, out_hbm,
                idxj_all, idxi_a, idxi_b, wij_a, wij_b, rows_a, rows_b,
                acc_sh, gsem_a, gsem_b, wsem_a, wsem_b, isem_a, isem_b):
        cid = lax.axis_index("c")
        sid = lax.axis_index("s")
        wid = sid * _NC + cid

        zed = jnp.zeros((16,), jnp.float32)

        def zero_body(r, _):
            for k in range(vecs_per_row):
                rows_a[r, pl.ds(k * 16, 16)] = zed
            return 0

        lax.fori_loop(0, chunk, zero_body, 0)

        row0 = sid * rows_per_tile
        done = 0
        while done < rows_per_tile:
            cnt = min(chunk, rows_per_tile - done)
            pltpu.sync_copy(rows_a.at[pl.ds(0, cnt)],
                            acc_sh.at[pl.ds(row0 + done, cnt)])
            done += cnt
        plsc.subcore_barrier()

        def run_half(wbase, ibase):
            pltpu.sync_copy(idxj_hbm.at[pl.ds(ibase, eph)], idxj_all)

            def start(ci, rows_v, wij_v, idxi_v, gsem, wsem, isem):
                pltpu.make_async_copy(
                    h_hbm.at[idxj_all.at[pl.ds(ci * chunk, chunk)]],
                    rows_v, gsem).start()
                pltpu.make_async_copy(
                    wij_hbm.at[pl.ds(wbase + ci * chunk, chunk)],
                    wij_v, wsem).start()
                pltpu.make_async_copy(
                    idxi_hbm.at[pl.ds(ibase + ci * chunk, chunk)],
                    idxi_v, isem).start()

            def wait(rows_v, wij_v, idxi_v, gsem, wsem, isem):
                pltpu.make_async_copy(h_hbm.at[pl.ds(0, chunk)],
                                      rows_v, gsem).wait()
                pltpu.make_async_copy(wij_hbm.at[pl.ds(0, chunk)],
                                      wij_v, wsem).wait()
                pltpu.make_async_copy(idxi_hbm.at[pl.ds(0, chunk)],
                                      idxi_v, isem).wait()

            def mul(rows_v, wij_v):
                @plsc.parallel_loop(0, chunk, 1, unroll=4)
                def row_body(r):
                    for k in range(vecs_per_row):
                        s = pl.ds(k * 16, 16)
                        rows_v[r, s] = rows_v[r, s] * wij_v[r, s]

            def scat(rows_v, idxi_v):
                pltpu.sync_copy(rows_v, acc_sh.at[idxi_v], add=True)

            start(0, rows_a, wij_a, idxi_a, gsem_a, wsem_a, isem_a)

            def pair_body(g, _):
                c0 = 2 * g
                start(c0 + 1, rows_b, wij_b, idxi_b,
                      gsem_b, wsem_b, isem_b)
                wait(rows_a, wij_a, idxi_a, gsem_a, wsem_a, isem_a)
                mul(rows_a, wij_a)
                scat(rows_a, idxi_a)
                start(c0 + 2, rows_a, wij_a, idxi_a,
                      gsem_a, wsem_a, isem_a)
                wait(rows_b, wij_b, idxi_b, gsem_b, wsem_b, isem_b)
                mul(rows_b, wij_b)
                scat(rows_b, idxi_b)
                return 0

            lax.fori_loop(0, nch // 2, pair_body, 0)
            wait(rows_a, wij_a, idxi_a, gsem_a, wsem_a, isem_a)
            mul(rows_a, wij_a)
            scat(rows_a, idxi_a)

        for hf in range(nhalf):
            run_half(wid * epw + hf * eph,
                     edge_off + wid * epw + hf * eph)
        plsc.subcore_barrier()

        pltpu.sync_copy(acc_sh.at[pl.ds(row0, rows_per_tile)],
                        out_hbm.at[cid, pl.ds(row0, rows_per_tile)])

    return sc_kern


def _out_body(p_ref, q_ref, w1_ref, b1_ref, w2_ref, b2_ref, o_ref):
    s = (p_ref[0] + p_ref[1]) + (q_ref[0] + q_ref[1])
    t = jnp.dot(s, w1_ref[...], preferred_element_type=jnp.float32)
    t = _ssp(t + b1_ref[...])
    t = jnp.dot(t, w2_ref[...], preferred_element_type=jnp.float32)
    o_ref[...] = t + b2_ref[...]


def _compute_out(p0, p1, n_out, W_o1, b_o1, W_o2, b_o2, block=1000):
    _, _, nf = p0.shape
    na = W_o1.shape[1]
    grid = n_out // block
    b1 = b_o1.reshape(1, na)
    b2 = b_o2.reshape(1, na)
    return pl.pallas_call(
        _out_body,
        grid=(grid,),
        in_specs=[
            pl.BlockSpec((2, block, nf), lambda i: (0, i, 0)),
            pl.BlockSpec((2, block, nf), lambda i: (0, i, 0)),
            pl.BlockSpec((nf, na), lambda i: (0, 0)),
            pl.BlockSpec((1, na), lambda i: (0, 0)),
            pl.BlockSpec((na, na), lambda i: (0, 0)),
            pl.BlockSpec((1, na), lambda i: (0, 0)),
        ],
        out_specs=pl.BlockSpec((block, na), lambda i: (i, 0)),
        out_shape=jax.ShapeDtypeStruct((n_out, na), jnp.float32),
    )(p0, p1, W_o1, b1, W_o2, b2)


def kernel(x, f_ij, idx_i, idx_j, rcut_ij,
           W_in2f, W_f1, b_f1, W_f2, b_f2, W_o1, b_o1, W_o2, b_o2):
    n_atoms, _ = x.shape
    n_edges, _ = f_ij.shape
    d = W_in2f.shape[1]

    h = _compute_h(x, W_in2f)
    idx_i = idx_i.astype(jnp.int32)
    idx_j = idx_j.astype(jnp.int32)

    e_slab = n_edges // 2
    wij0 = _compute_wij(f_ij, rcut_ij, W_f1, b_f1, W_f2, b_f2, e_slab, 0)
    sc0 = _make_sc_gather_scatter(n_atoms, e_slab, 0, d)
    p0 = sc0(h, wij0, idx_i, idx_j)
    wij1 = _compute_wij(f_ij, rcut_ij, W_f1, b_f1, W_f2, b_f2,
                        e_slab, e_slab)
    sc1 = _make_sc_gather_scatter(n_atoms, e_slab, e_slab, d)
    p1 = sc1(h, wij1, idx_i, idx_j)

    return _compute_out(p0, p1, n_atoms, W_o1, b_o1, W_o2, b_o2)


# speedup vs baseline: 1.3117x; 1.3117x over previous
import functools

import jax
import jax.numpy as jnp
from jax import lax
from jax.experimental import pallas as pl
from jax.experimental.pallas import tpu as pltpu
from jax.experimental.pallas import tpu_sc as plsc

_LOG2 = 0.6931471805599453

_NC = 2
_NS = 16
_NW = _NC * _NS


def _ssp(x):
    return jax.nn.softplus(x) - _LOG2


def _h_body(x_ref, w_ref, o_ref):
    o_ref[...] = jnp.dot(x_ref[...], w_ref[...],
                         preferred_element_type=jnp.float32)


def _compute_h(x, W_in2f, block=1000):
    n, d = x.shape
    f = W_in2f.shape[1]
    grid = n // block
    return pl.pallas_call(
        _h_body,
        grid=(grid,),
        in_specs=[
            pl.BlockSpec((block, d), lambda i: (i, 0)),
            pl.BlockSpec((d, f), lambda i: (0, 0)),
        ],
        out_specs=pl.BlockSpec((block, f), lambda i: (i, 0)),
        out_shape=jax.ShapeDtypeStruct((n, f), jnp.float32),
    )(x, W_in2f)


def _filter_body(f_ref, rc_ref, w1_ref, b1_ref, w2_ref, b2_ref, o_ref):
    t = jnp.dot(f_ref[...], w1_ref[...], preferred_element_type=jnp.float32)
    t = _ssp(t + b1_ref[...])
    t = jnp.dot(t, w2_ref[...], preferred_element_type=jnp.float32)
    rc = jnp.transpose(rc_ref[0])
    o_ref[...] = (t + b2_ref[...]) * rc


def _compute_wij(f_ij, rcut, W_f1, b_f1, W_f2, b_f2, e_slab, row_off,
                 block=4000):
    e, r = f_ij.shape
    nf = W_f1.shape[1]
    grid = e_slab // block
    off = row_off // block
    b1 = b_f1.reshape(1, nf)
    b2 = b_f2.reshape(1, nf)
    return pl.pallas_call(
        _filter_body,
        grid=(grid,),
        in_specs=[
            pl.BlockSpec((block, r), lambda i: (i + off, 0)),
            pl.BlockSpec((1, 1, block), lambda i: (i + off, 0, 0)),
            pl.BlockSpec((r, nf), lambda i: (0, 0)),
            pl.BlockSpec((1, nf), lambda i: (0, 0)),
            pl.BlockSpec((nf, nf), lambda i: (0, 0)),
            pl.BlockSpec((1, nf), lambda i: (0, 0)),
        ],
        out_specs=pl.BlockSpec((block, nf), lambda i: (i, 0)),
        out_shape=jax.ShapeDtypeStruct((e_slab, nf), jnp.float32),
    )(f_ij, rcut.reshape(e // block, 1, block), W_f1, b1, W_f2, b2)


def _make_sc_gather_scatter(n_atoms, n_slab, edge_off, d, chunk=40,
                            nhalf=1):
    epw = n_slab // _NW
    nchunk = epw // chunk
    nch = nchunk // nhalf
    eph = nch * chunk
    assert nchunk == nch * nhalf and nch % 2 == 1 and nch >= 3
    n_pad = ((n_atoms + 8 * _NS - 1) // (8 * _NS)) * (8 * _NS)
    rows_per_tile = n_pad // _NS
    vecs_per_row = d // 16
    mesh = plsc.VectorSubcoreMesh(core_axis_name="c", subcore_axis_name="s")

    @functools.partial(
        pl.kernel,
        out_type=jax.ShapeDtypeStruct((_NC, n_pad, d), jnp.float32),
        mesh=mesh,
        scratch_types=[
            pltpu.VMEM((eph,), jnp.int32),
            pltpu.VMEM((chunk,), jnp.int32),
            pltpu.VMEM((chunk,), jnp.int32),
            pltpu.VMEM((chunk, d), jnp.float32),
            pltpu.VMEM((chunk, d), jnp.float32),
            pltpu.VMEM((chunk, d), jnp.float32),
            pltpu.VMEM((chunk, d), jnp.float32),
            pltpu.VMEM_SHARED((n_pad, d), jnp.float32),
            pltpu.SemaphoreType.DMA,
            pltpu.SemaphoreType.DMA,
            pltpu.SemaphoreType.DMA,
            pltpu.SemaphoreType.DMA,
            pltpu.SemaphoreType.DMA,
            pltpu.SemaphoreType.DMA,
        ],
    )
    def sc_kern(h_hbm, wij_hbm, idxi_hbm, idxj_hbm, out_hbm,
                idxj_all, idxi_a, idxi_b, wij_a, wij_b, rows_a, rows_b,
                acc_sh, gsem_a, gsem_b, wsem_a, wsem_b, isem_a, isem_b):
        cid = lax.axis_index("c")
        sid = lax.axis_index("s")
        wid = sid * _NC + cid

        zed = jnp.zeros((16,), jnp.float32)

        def zero_body(r, _):
            for k in range(vecs_per_row):
                rows_a[r, pl.ds(k * 16, 16)] = zed
            return 0

        lax.fori_loop(0, chunk, zero_body, 0)

        row0 = sid * rows_per_tile
        done = 0
        while done < rows_per_tile:
            cnt = min(chunk, rows_per_tile - done)
            pltpu.sync_copy(rows_a.at[pl.ds(0, cnt)],
                            acc_sh.at[pl.ds(row0 + done, cnt)])
            done += cnt
        plsc.subcore_barrier()

        def run_half(wbase, ibase):
            pltpu.sync_copy(idxj_hbm.at[pl.ds(ibase, eph)], idxj_all)

            def start(ci, rows_v, wij_v, idxi_v, gsem, wsem, isem):
                pltpu.make_async_copy(
                    h_hbm.at[idxj_all.at[pl.ds(ci * chunk, chunk)]],
                    rows_v, gsem).start()
                pltpu.make_async_copy(
                    wij_hbm.at[pl.ds(wbase + ci * chunk, chunk)],
                    wij_v, wsem).start()
                pltpu.make_async_copy(
                    idxi_hbm.at[pl.ds(ibase + ci * chunk, chunk)],
                    idxi_v, isem).start()

            def wait(rows_v, wij_v, idxi_v, gsem, wsem, isem):
                pltpu.make_async_copy(h_hbm.at[pl.ds(0, chunk)],
                                      rows_v, gsem).wait()
                pltpu.make_async_copy(wij_hbm.at[pl.ds(0, chunk)],
                                      wij_v, wsem).wait()
                pltpu.make_async_copy(idxi_hbm.at[pl.ds(0, chunk)],
                                      idxi_v, isem).wait()

            def mul(rows_v, wij_v):
                @plsc.parallel_loop(0, chunk, 1, unroll=4)
                def row_body(r):
                    for k in range(vecs_per_row):
                        s = pl.ds(k * 16, 16)
                        rows_v[r, s] = rows_v[r, s] * wij_v[r, s]

            def scat(rows_v, idxi_v):
                pltpu.sync_copy(rows_v, acc_sh.at[idxi_v], add=True)

            start(0, rows_a, wij_a, idxi_a, gsem_a, wsem_a, isem_a)

            def pair_body(g, _):
                c0 = 2 * g
                start(c0 + 1, rows_b, wij_b, idxi_b,
                      gsem_b, wsem_b, isem_b)
                wait(rows_a, wij_a, idxi_a, gsem_a, wsem_a, isem_a)
                mul(rows_a, wij_a)
                scat(rows_a, idxi_a)
                start(c0 + 2, rows_a, wij_a, idxi_a,
                      gsem_a, wsem_a, isem_a)
                wait(rows_b, wij_b, idxi_b, gsem_b, wsem_b, isem_b)
                mul(rows_b, wij_b)
                scat(rows_b, idxi_b)
                return 0

            lax.fori_loop(0, nch // 2, pair_body, 0)
            wait(rows_a, wij_a, idxi_a, gsem_a, wsem_a, isem_a)
            mul(rows_a, wij_a)
            scat(rows_a, idxi_a)

        for hf in range(nhalf):
            run_half(wid * epw + hf * eph,
                     edge_off + wid * epw + hf * eph)
        plsc.subcore_barrier()

        pltpu.sync_copy(acc_sh.at[pl.ds(row0, rows_per_tile)],
                        out_hbm.at[cid, pl.ds(row0, rows_per_tile)])

    return sc_kern


def _out_body(p_ref, q_ref, w1_ref, b1_ref, w2_ref, b2_ref, o_ref):
    s = (p_ref[0] + p_ref[1]) + (q_ref[0] + q_ref[1])
    t = jnp.dot(s, w1_ref[...], preferred_element_type=jnp.float32)
    t = _ssp(t + b1_ref[...])
    t = jnp.dot(t, w2_ref[...], preferred_element_type=jnp.float32)
    o_ref[...] = t + b2_ref[...]


def _compute_out(p0, p1, n_out, W_o1, b_o1, W_o2, b_o2, block=1000):
    _, _, nf = p0.shape
    na = W_o1.shape[1]
    grid = n_out // block
    b1 = b_o1.reshape(1, na)
    b2 = b_o2.reshape(1, na)
    return pl.pallas_call(
        _out_body,
        grid=(grid,),
        in_specs=[
            pl.BlockSpec((2, block, nf), lambda i: (0, i, 0)),
            pl.BlockSpec((2, block, nf), lambda i: (0, i, 0)),
            pl.BlockSpec((nf, na), lambda i: (0, 0)),
            pl.BlockSpec((1, na), lambda i: (0, 0)),
            pl.BlockSpec((na, na), lambda i: (0, 0)),
            pl.BlockSpec((1, na), lambda i: (0, 0)),
        ],
        out_specs=pl.BlockSpec((block, na), lambda i: (i, 0)),
        out_shape=jax.ShapeDtypeStruct((n_out, na), jnp.float32),
    )(p0, p1, W_o1, b1, W_o2, b2)


def kernel(x, f_ij, idx_i, idx_j, rcut_ij,
           W_in2f, W_f1, b_f1, W_f2, b_f2, W_o1, b_o1, W_o2, b_o2):
    n_atoms, _ = x.shape
    n_edges, _ = f_ij.shape
    d = W_in2f.shape[1]

    h = _compute_h(x, W_in2f)
    idx_i = idx_i.astype(jnp.int32)
    idx_j = idx_j.astype(jnp.int32)

    e_slab = n_edges // 2
    wij0 = _compute_wij(f_ij, rcut_ij, W_f1, b_f1, W_f2, b_f2, e_slab, 0)
    sc0 = _make_sc_gather_scatter(n_atoms, e_slab, 0, d)
    p0 = sc0(h, wij0, idx_i, idx_j)
    wij1 = _compute_wij(f_ij, rcut_ij, W_f1, b_f1, W_f2, b_f2,
                        e_slab, e_slab)
    sc1 = _make_sc_gather_scatter(n_atoms, e_slab, e_slab, d)
    p1 = sc1(h, wij1, idx_i, idx_j)

    return _compute_out(p0, p1, n_atoms, W_o1, b_o1, W_o2, b_o2)


# speedup vs baseline: 1.6538x; 1.2608x over previous
import functools

import jax
import jax.numpy as jnp
from jax import lax
from jax.experimental import pallas as pl
from jax.experimental.pallas import tpu as pltpu
from jax.experimental.pallas import tpu_sc as plsc

_LOG2 = 0.6931471805599453

_NC = 2
_NS = 16
_NW = _NC * _NS


def _ssp(x):
    return jax.nn.softplus(x) - _LOG2


def _h_body(x_ref, w_ref, o_ref):
    o_ref[...] = jnp.dot(x_ref[...], w_ref[...],
                         preferred_element_type=jnp.float32)


def _compute_h(x, W_in2f, block=1000):
    n, d = x.shape
    f = W_in2f.shape[1]
    grid = n // block
    return pl.pallas_call(
        _h_body,
        grid=(grid,),
        in_specs=[
            pl.BlockSpec((block, d), lambda i: (i, 0)),
            pl.BlockSpec((d, f), lambda i: (0, 0)),
        ],
        out_specs=pl.BlockSpec((block, f), lambda i: (i, 0)),
        out_shape=jax.ShapeDtypeStruct((n, f), jnp.float32),
    )(x, W_in2f)


def _filter_body(f_ref, rc_ref, w1_ref, b1_ref, w2_ref, b2_ref, o_ref):
    t = lax.dot_general(f_ref[...], w1_ref[...],
                        dimension_numbers=(((0,), (0,)), ((), ())),
                        preferred_element_type=jnp.float32)
    t = _ssp(t + b1_ref[...])
    t = jnp.dot(t, w2_ref[...], preferred_element_type=jnp.float32)
    rc = jnp.transpose(rc_ref[0])
    o_ref[...] = (t + b2_ref[...]) * rc


def _compute_wij(f_ij, rcut, W_f1, b_f1, W_f2, b_f2, e_slab, row_off,
                 block=3200):
    e, r = f_ij.shape
    nf = W_f1.shape[1]
    grid = e_slab // block
    off = row_off // block
    b1 = b_f1.reshape(1, nf)
    b2 = b_f2.reshape(1, nf)
    return pl.pallas_call(
        _filter_body,
        grid=(grid,),
        in_specs=[
            pl.BlockSpec((r, block), lambda i: (0, i + off)),
            pl.BlockSpec((1, 1, block), lambda i: (i + off, 0, 0)),
            pl.BlockSpec((r, nf), lambda i: (0, 0)),
            pl.BlockSpec((1, nf), lambda i: (0, 0)),
            pl.BlockSpec((nf, nf), lambda i: (0, 0)),
            pl.BlockSpec((1, nf), lambda i: (0, 0)),
        ],
        out_specs=pl.BlockSpec((block, nf), lambda i: (i, 0)),
        out_shape=jax.ShapeDtypeStruct((e_slab, nf), jnp.float32),
    )(f_ij.T, rcut.reshape(e // block, 1, block), W_f1, b1, W_f2, b2)


def _make_sc_gather_scatter(n_atoms, n_slab, edge_off, d, chunk=40,
                            nhalf=1):
    epw = n_slab // _NW
    nchunk = epw // chunk
    nch = nchunk // nhalf
    eph = nch * chunk
    assert nchunk == nch * nhalf and nch % 2 == 1 and nch >= 3
    n_pad = ((n_atoms + 8 * _NS - 1) // (8 * _NS)) * (8 * _NS)
    rows_per_tile = n_pad // _NS
    vecs_per_row = d // 16
    mesh = plsc.VectorSubcoreMesh(core_axis_name="c", subcore_axis_name="s")

    @functools.partial(
        pl.kernel,
        out_type=jax.ShapeDtypeStruct((_NC, n_pad, d), jnp.float32),
        mesh=mesh,
        scratch_types=[
            pltpu.VMEM((eph,), jnp.int32),
            pltpu.VMEM((chunk,), jnp.int32),
            pltpu.VMEM((chunk,), jnp.int32),
            pltpu.VMEM((chunk, d), jnp.float32),
            pltpu.VMEM((chunk, d), jnp.float32),
            pltpu.VMEM((chunk, d), jnp.float32),
            pltpu.VMEM((chunk, d), jnp.float32),
            pltpu.VMEM_SHARED((n_pad, d), jnp.float32),
            pltpu.SemaphoreType.DMA,
            pltpu.SemaphoreType.DMA,
            pltpu.SemaphoreType.DMA,
            pltpu.SemaphoreType.DMA,
            pltpu.SemaphoreType.DMA,
            pltpu.SemaphoreType.DMA,
        ],
    )
    def sc_kern(h_hbm, wij_hbm, idxi_hbm, idxj_hbm, out_hbm,
                idxj_all, idxi_a, idxi_b, wij_a, wij_b, rows_a, rows_b,
                acc_sh, gsem_a, gsem_b, wsem_a, wsem_b, isem_a, isem_b):
        cid = lax.axis_index("c")
        sid = lax.axis_index("s")
        wid = sid * _NC + cid

        zed = jnp.zeros((16,), jnp.float32)

        def zero_body(r, _):
            for k in range(vecs_per_row):
                rows_a[r, pl.ds(k * 16, 16)] = zed
            return 0

        lax.fori_loop(0, chunk, zero_body, 0)

        row0 = sid * rows_per_tile
        done = 0
        while done < rows_per_tile:
            cnt = min(chunk, rows_per_tile - done)
            pltpu.sync_copy(rows_a.at[pl.ds(0, cnt)],
                            acc_sh.at[pl.ds(row0 + done, cnt)])
            done += cnt
        plsc.subcore_barrier()

        def run_half(wbase, ibase):
            pltpu.sync_copy(idxj_hbm.at[pl.ds(ibase, eph)], idxj_all)

            def start(ci, rows_v, wij_v, idxi_v, gsem, wsem, isem):
                pltpu.make_async_copy(
                    h_hbm.at[idxj_all.at[pl.ds(ci * chunk, chunk)]],
                    rows_v, gsem).start()
                pltpu.make_async_copy(
                    wij_hbm.at[pl.ds(wbase + ci * chunk, chunk)],
                    wij_v, wsem).start()
                pltpu.make_async_copy(
                    idxi_hbm.at[pl.ds(ibase + ci * chunk, chunk)],
                    idxi_v, isem).start()

            def wait(rows_v, wij_v, idxi_v, gsem, wsem, isem):
                pltpu.make_async_copy(h_hbm.at[pl.ds(0, chunk)],
                                      rows_v, gsem).wait()
                pltpu.make_async_copy(wij_hbm.at[pl.ds(0, chunk)],
                                      wij_v, wsem).wait()
                pltpu.make_async_copy(idxi_hbm.at[pl.ds(0, chunk)],
                                      idxi_v, isem).wait()

            def mul(rows_v, wij_v):
                @plsc.parallel_loop(0, chunk, 1, unroll=4)
                def row_body(r):
                    for k in range(vecs_per_row):
                        s = pl.ds(k * 16, 16)
                        rows_v[r, s] = rows_v[r, s] * wij_v[r, s]

            def scat(rows_v, idxi_v):
                pltpu.sync_copy(rows_v, acc_sh.at[idxi_v], add=True)

            start(0, rows_a, wij_a, idxi_a, gsem_a, wsem_a, isem_a)

            def pair_body(g, _):
                c0 = 2 * g
                start(c0 + 1, rows_b, wij_b, idxi_b,
                      gsem_b, wsem_b, isem_b)
                wait(rows_a, wij_a, idxi_a, gsem_a, wsem_a, isem_a)
                mul(rows_a, wij_a)
                scat(rows_a, idxi_a)
                start(c0 + 2, rows_a, wij_a, idxi_a,
                      gsem_a, wsem_a, isem_a)
                wait(rows_b, wij_b, idxi_b, gsem_b, wsem_b, isem_b)
                mul(rows_b, wij_b)
                scat(rows_b, idxi_b)
                return 0

            lax.fori_loop(0, nch // 2, pair_body, 0)
            wait(rows_a, wij_a, idxi_a, gsem_a, wsem_a, isem_a)
            mul(rows_a, wij_a)
            scat(rows_a, idxi_a)

        for hf in range(nhalf):
            run_half(wid * epw + hf * eph,
                     edge_off + wid * epw + hf * eph)
        plsc.subcore_barrier()

        pltpu.sync_copy(acc_sh.at[pl.ds(row0, rows_per_tile)],
                        out_hbm.at[cid, pl.ds(row0, rows_per_tile)])

    return sc_kern


def _out_body(p_ref, q_ref, w1_ref, b1_ref, w2_ref, b2_ref, o_ref):
    s = (p_ref[0] + p_ref[1]) + (q_ref[0] + q_ref[1])
    t = jnp.dot(s, w1_ref[...], preferred_element_type=jnp.float32)
    t = _ssp(t + b1_ref[...])
    t = jnp.dot(t, w2_ref[...], preferred_element_type=jnp.float32)
    o_ref[...] = t + b2_ref[...]


def _compute_out(p0, p1, n_out, W_o1, b_o1, W_o2, b_o2, block=1000):
    _, _, nf = p0.shape
    na = W_o1.shape[1]
    grid = n_out // block
    b1 = b_o1.reshape(1, na)
    b2 = b_o2.reshape(1, na)
    return pl.pallas_call(
        _out_body,
        grid=(grid,),
        in_specs=[
            pl.BlockSpec((2, block, nf), lambda i: (0, i, 0)),
            pl.BlockSpec((2, block, nf), lambda i: (0, i, 0)),
            pl.BlockSpec((nf, na), lambda i: (0, 0)),
            pl.BlockSpec((1, na), lambda i: (0, 0)),
            pl.BlockSpec((na, na), lambda i: (0, 0)),
            pl.BlockSpec((1, na), lambda i: (0, 0)),
        ],
        out_specs=pl.BlockSpec((block, na), lambda i: (i, 0)),
        out_shape=jax.ShapeDtypeStruct((n_out, na), jnp.float32),
    )(p0, p1, W_o1, b1, W_o2, b2)


def kernel(x, f_ij, idx_i, idx_j, rcut_ij,
           W_in2f, W_f1, b_f1, W_f2, b_f2, W_o1, b_o1, W_o2, b_o2):
    n_atoms, _ = x.shape
    n_edges, _ = f_ij.shape
    d = W_in2f.shape[1]

    h = _compute_h(x, W_in2f)
    idx_i = idx_i.astype(jnp.int32)
    idx_j = idx_j.astype(jnp.int32)

    e_slab = n_edges // 2
    wij0 = _compute_wij(f_ij, rcut_ij, W_f1, b_f1, W_f2, b_f2, e_slab, 0)
    sc0 = _make_sc_gather_scatter(n_atoms, e_slab, 0, d)
    p0 = sc0(h, wij0, idx_i, idx_j)
    wij1 = _compute_wij(f_ij, rcut_ij, W_f1, b_f1, W_f2, b_f2,
                        e_slab, e_slab)
    sc1 = _make_sc_gather_scatter(n_atoms, e_slab, e_slab, d)
    p1 = sc1(h, wij1, idx_i, idx_j)

    return _compute_out(p0, p1, n_atoms, W_o1, b_o1, W_o2, b_o2)
